# baseline (speedup 1.0000x reference)
import functools

import jax
import jax.numpy as jnp
from jax import lax
from jax.experimental import pallas as pl
from jax.experimental.pallas import tpu as pltpu
from jax.experimental.pallas import tpu_sc as plsc

_MAXLEN = 2048
_HEAD_DIM = 32
_TROWS = 2 * _MAXLEN
_NPHASE = 128
_PADW = _TROWS + _NPHASE

_info = plsc.get_sparse_core_info()
_NC = _info.num_cores


_PPB = 8


def _phase_body(stab_ref, ph_ref):
    pid = pl.program_id(0)
    s = stab_ref[...]
    for u in range(_PPB):
        phi = pid * _PPB + u
        r = pltpu.roll(s, _PADW - phi, axis=1)
        ph_ref[u] = jnp.minimum(jnp.maximum(r[:, :_TROWS], -5.0), 5.0)


_phase_tc = pl.pallas_call(
    _phase_body,
    grid=(_NPHASE // _PPB,),
    in_specs=[pl.BlockSpec((_HEAD_DIM, _PADW), lambda p: (0, 0))],
    out_specs=pl.BlockSpec((_PPB, _HEAD_DIM, _TROWS), lambda p: (p, 0, 0)),
    out_shape=jax.ShapeDtypeStruct((_NPHASE, _HEAD_DIM, _TROWS), jnp.float32),
)

_GRP = 2
_NBAT = _NPHASE // _GRP


@functools.partial(
    pl.kernel,
    mesh=plsc.ScalarSubcoreMesh(axis_name="c", num_cores=_NC),
    out_type=jax.ShapeDtypeStruct((_MAXLEN, _HEAD_DIM, _MAXLEN), jnp.float32),
    scratch_types=[
        pltpu.VMEM_SHARED((4, _GRP, _HEAD_DIM, _TROWS), jnp.float32),
        pltpu.SemaphoreType.DMA,
        pltpu.SemaphoreType.DMA,
    ],
)
def _writeout_scs(ph_hbm, out_hbm, spmem, stage_sem, row_sem):
    cid = lax.axis_index("c")

    def _stage(b):
        return pltpu.make_async_copy(
            ph_hbm.at[pl.ds(b * _GRP, _GRP)],
            spmem.at[lax.rem(b, 4)],
            stage_sem,
        )

    def _copy(b, g, t):
        phi = b * _GRP + g
        kb = (1 - cid) * 8 + jnp.where(phi == 0, 1, 0)
        k0 = kb + t
        i = _MAXLEN - phi - 128 * k0
        return pltpu.make_async_copy(
            spmem.at[lax.rem(b, 4), g, :, pl.ds(128 * k0, _MAXLEN)],
            out_hbm.at[i],
            row_sem,
        )

    def _fire_rows(b):
        for g in range(_GRP):
            for t in range(8):
                _copy(b, g, t).start()

    def _drain_rows(b):
        for g in range(_GRP):
            for t in range(8):
                _copy(b, g, t).wait()

    _stage(0).start()
    _stage(1).start()

    def batch_body(b, _):
        _stage(b).wait()

        @pl.when(b >= 2)
        def _():
            _drain_rows(b - 2)

        @pl.when(b + 2 < _NBAT)
        def _():
            _stage(b + 2).start()

        _fire_rows(b)
        return 0

    lax.fori_loop(0, _NBAT, batch_body, 0)
    _drain_rows(_NBAT - 2)
    _drain_rows(_NBAT - 1)


_NW = _NC * _info.num_subcores
_ROWS_PER_W = _MAXLEN // _NW


@functools.partial(
    pl.kernel,
    mesh=plsc.VectorSubcoreMesh(core_axis_name="c", subcore_axis_name="s"),
    out_type=jax.ShapeDtypeStruct((_MAXLEN, _HEAD_DIM, _MAXLEN), jnp.float32),
    scratch_types=[
        pltpu.VMEM((8, _MAXLEN), jnp.float32),
    ],
)
def _writeout_tec(ph_hbm, out_hbm, vbuf):
    cid = lax.axis_index("c")
    sid = lax.axis_index("s")
    wid = sid * _NC + cid
    row0 = wid * _ROWS_PER_W

    def row_body(r, _):
        i = row0 + r
        c = _MAXLEN - i
        phi = lax.rem(c, 128)
        k0 = lax.div(c, 128)
        for td in range(4):
            pltpu.sync_copy(
                ph_hbm.at[phi, pl.ds(8 * td, 8), pl.ds(128 * k0, _MAXLEN)],
                vbuf,
            )
            pltpu.sync_copy(vbuf, out_hbm.at[i, pl.ds(8 * td, 8), :])
        return 0

    lax.fori_loop(0, _ROWS_PER_W, row_body, 0)


def kernel(pe_k_weight, seq_len):
    del seq_len
    stab = jnp.pad(jnp.transpose(pe_k_weight), ((0, 0), (0, _NPHASE)))
    ph = _phase_tc(stab)
    p = _writeout_tec(ph)
    return jnp.transpose(p, (0, 2, 1))


# speedup vs baseline: 1.8429x; 1.8429x over previous
import functools

import jax
import jax.numpy as jnp
from jax import lax
from jax.experimental import pallas as pl
from jax.experimental.pallas import tpu as pltpu
from jax.experimental.pallas import tpu_sc as plsc

_MAXLEN = 2048
_HEAD_DIM = 32
_TROWS = 2 * _MAXLEN
_NPHASE = 128
_PADW = _TROWS + _NPHASE

_info = plsc.get_sparse_core_info()
_NC = _info.num_cores
_NS = _info.num_subcores
_NW = _NC * _NS


_PPB = 8


def _phase_body(stab_ref, ph_ref):
    pid = pl.program_id(0)
    s = stab_ref[...]
    for u in range(_PPB):
        phi = pid * _PPB + u
        r = pltpu.roll(s, _PADW - phi, axis=1)
        ph_ref[u] = jnp.minimum(jnp.maximum(r[:, :_TROWS], -5.0), 5.0)


_phase_tc = pl.pallas_call(
    _phase_body,
    grid=(_NPHASE // _PPB,),
    in_specs=[pl.BlockSpec((_HEAD_DIM, _PADW), lambda p: (0, 0))],
    out_specs=pl.BlockSpec((_PPB, _HEAD_DIM, _TROWS), lambda p: (p, 0, 0)),
    out_shape=jax.ShapeDtypeStruct((_NPHASE, _HEAD_DIM, _TROWS), jnp.float32),
)

_GRP = 2
_NBAT = _NPHASE // _GRP

_TEC_ROWS = 639
_TEC_PER_W = 20


def _writeout_tec(ph_hbm, out_hbm, vbuf, spmem, stage_sem, row_sem):
    del spmem, stage_sem, row_sem
    cid = lax.axis_index("c")
    sid = lax.axis_index("s")
    wid = sid * _NC + cid
    n = jnp.where(wid == _NW - 1, _TEC_ROWS - _TEC_PER_W * (_NW - 1),
                  _TEC_PER_W)

    def row_body(r, _):
        c = 1 + wid * _TEC_PER_W + r
        i = _MAXLEN - c
        phi = lax.rem(c, 128)
        k0 = lax.div(c, 128)
        for td in range(4):
            pltpu.sync_copy(
                ph_hbm.at[phi, pl.ds(8 * td, 8), pl.ds(128 * k0, _MAXLEN)],
                vbuf,
            )
            pltpu.sync_copy(vbuf, out_hbm.at[i, pl.ds(8 * td, 8), :])
        return 0

    lax.fori_loop(0, n, row_body, 0)


def _writeout_scs(ph_hbm, out_hbm, vbuf, spmem, stage_sem, row_sem):
    del vbuf
    cid = lax.axis_index("c")
    kb = jnp.where(cid == 0, 11, 5)
    nk = jnp.where(cid == 0, 5, 6)

    def _stage(b):
        return pltpu.make_async_copy(
            ph_hbm.at[pl.ds(b * _GRP, _GRP)],
            spmem.at[lax.rem(b, 4)],
            stage_sem,
        )

    def _copy_k(b, g, k0):
        phi = b * _GRP + g
        i = _MAXLEN - phi - 128 * k0
        return pltpu.make_async_copy(
            spmem.at[lax.rem(b, 4), g, :, pl.ds(128 * k0, _MAXLEN)],
            out_hbm.at[i],
            row_sem,
        )

    def _rows(b, fire):
        for g in range(_GRP):
            for t in range(6):
                @pl.when(t < nk)
                def _():
                    if fire:
                        _copy_k(b, g, kb + t).start()
                    else:
                        _copy_k(b, g, kb + t).wait()
            @pl.when(jnp.logical_and(b * _GRP + g == 0, cid == 0))
            def _():
                if fire:
                    _copy_k(b, g, 16).start()
                else:
                    _copy_k(b, g, 16).wait()

    _stage(0).start()
    _stage(1).start()

    def batch_body(b, _):
        _stage(b).wait()

        @pl.when(b >= 2)
        def _():
            _rows(b - 2, False)

        @pl.when(b + 2 < _NBAT)
        def _():
            _stage(b + 2).start()

        _rows(b, True)
        return 0

    lax.fori_loop(0, _NBAT, batch_body, 0)
    _rows(_NBAT - 2, False)
    _rows(_NBAT - 1, False)


_vmesh = plsc.VectorSubcoreMesh(core_axis_name="c", subcore_axis_name="s")
_smesh = plsc.ScalarSubcoreMesh(axis_name="c", num_cores=_NC)

_writeout_sc = pl.kernel(
    [_writeout_tec, _writeout_scs],
    out_type=jax.ShapeDtypeStruct((_MAXLEN, _HEAD_DIM, _MAXLEN), jnp.float32),
    mesh=[_vmesh, _smesh],
    scratch_types=[
        pltpu.VMEM((8, _MAXLEN), jnp.float32) @ _vmesh,
        pltpu.VMEM_SHARED((4, _GRP, _HEAD_DIM, _TROWS), jnp.float32),
        pltpu.SemaphoreType.DMA @ _smesh,
        pltpu.SemaphoreType.DMA @ _smesh,
    ],
)


def kernel(pe_k_weight, seq_len):
    del seq_len
    stab = jnp.pad(jnp.transpose(pe_k_weight), ((0, 0), (0, _NPHASE)))
    ph = _phase_tc(stab)
    p = _writeout_sc(ph)
    return jnp.transpose(p, (0, 2, 1))


# speedup vs baseline: 1.9699x; 1.0689x over previous
import functools

import jax
import jax.numpy as jnp
from jax import lax
from jax.experimental import pallas as pl
from jax.experimental.pallas import tpu as pltpu
from jax.experimental.pallas import tpu_sc as plsc

_MAXLEN = 2048
_HEAD_DIM = 32
_TROWS = 2 * _MAXLEN
_NPHASE = 128
_PADW = _TROWS + _NPHASE

_info = plsc.get_sparse_core_info()
_NC = _info.num_cores


_PPB = 8


def _phase_body(stab_ref, ph_ref):
    pid = pl.program_id(0)
    s = stab_ref[...]
    for u in range(_PPB):
        phi = pid * _PPB + u
        r = pltpu.roll(s, _PADW - phi, axis=1)
        ph_ref[u] = jnp.minimum(jnp.maximum(r[:, :_TROWS], -5.0), 5.0)


_phase_tc = pl.pallas_call(
    _phase_body,
    grid=(_NPHASE // _PPB,),
    in_specs=[pl.BlockSpec((_HEAD_DIM, _PADW), lambda p: (0, 0))],
    out_specs=pl.BlockSpec((_PPB, _HEAD_DIM, _TROWS), lambda p: (p, 0, 0)),
    out_shape=jax.ShapeDtypeStruct((_NPHASE, _HEAD_DIM, _TROWS), jnp.float32),
)

_GRP = 2
_NBAT = _NPHASE // _GRP


@functools.partial(
    pl.kernel,
    mesh=plsc.ScalarSubcoreMesh(axis_name="c", num_cores=_NC),
    out_type=jax.ShapeDtypeStruct((_MAXLEN, _HEAD_DIM, _MAXLEN), jnp.float32),
    scratch_types=[
        pltpu.VMEM_SHARED((4, _GRP, _HEAD_DIM, _TROWS), jnp.float32),
        pltpu.SemaphoreType.DMA,
        pltpu.SemaphoreType.DMA,
    ],
)
def _writeout_scs(ph_hbm, out_hbm, spmem, stage_sem, row_sem):
    cid = lax.axis_index("c")

    def _stage(b):
        return pltpu.make_async_copy(
            ph_hbm.at[pl.ds(b * _GRP, _GRP)],
            spmem.at[lax.rem(b, 4)],
            stage_sem,
        )

    def _copy(b, g, t):
        phi = b * _GRP + g
        kb = (1 - cid) * 8 + jnp.where(phi == 0, 1, 0)
        k0 = kb + t
        i = _MAXLEN - phi - 128 * k0
        return pltpu.make_async_copy(
            spmem.at[lax.rem(b, 4), g, :, pl.ds(128 * k0, _MAXLEN)],
            out_hbm.at[i],
            row_sem,
        )

    def _fire_rows(b):
        for g in range(_GRP):
            for t in range(8):
                _copy(b, g, t).start()

    def _drain_rows(b):
        for g in range(_GRP):
            for t in range(8):
                _copy(b, g, t).wait()

    _stage(0).start()
    _stage(1).start()

    def batch_body(b, _):
        _stage(b).wait()

        @pl.when(b >= 2)
        def _():
            _drain_rows(b - 2)

        @pl.when(b + 2 < _NBAT)
        def _():
            _stage(b + 2).start()

        _fire_rows(b)
        return 0

    lax.fori_loop(0, _NBAT, batch_body, 0)
    _drain_rows(_NBAT - 2)
    _drain_rows(_NBAT - 1)


def kernel(pe_k_weight, seq_len):
    del seq_len
    stab = jnp.pad(jnp.transpose(pe_k_weight), ((0, 0), (0, _NPHASE)))
    ph = _phase_tc(stab)
    p = _writeout_scs(ph)
    return jnp.transpose(p, (0, 2, 1))
